# baseline (speedup 1.0000x reference)
import functools

import jax
import jax.numpy as jnp
from jax import lax
from jax.experimental import pallas as pl
from jax.experimental.pallas import tpu as pltpu
from jax.experimental.pallas import tpu_sc as plsc

_N, _E, _D, _H, _O, _Q = 10000, 320000, 128, 128, 2, 4096
_NC, _NS = 2, 16
_NW = _NC * _NS
_EPW = _E // _NW
_CH = 120
_EPWP = 10080
_NCH = _EPWP // _CH
_BCH = 12
_NBLK = _NCH // _BCH
_NPAD = _N + 16
_RB = 624
_RREM = _N - _NS * _RB
_QW = 2 * _Q // _NW
_QCH = _QW // 128

_sc_mesh = plsc.VectorSubcoreMesh(core_axis_name="c", subcore_axis_name="s")


@functools.partial(
    pl.kernel,
    out_type=jax.ShapeDtypeStruct((_NC * _N, _H), jnp.float32),
    mesh=_sc_mesh,
    scratch_types=[
        pltpu.VMEM((2, _BCH, 2, _CH), jnp.int32),
        pltpu.VMEM((_CH, _H), jnp.float32),
        pltpu.VMEM((_CH, _H), jnp.float32),
        pltpu.VMEM_SHARED((_NPAD, _H), jnp.float32),
        pltpu.SemaphoreType.DMA,
        pltpu.SemaphoreType.DMA,
        pltpu.SemaphoreType.DMA,
        pltpu.SemaphoreType.DMA,
    ],
)
def _spmm(m_hbm, eidx_hbm, zeros_hbm, part_hbm,
          cidx_v, rows0_v, rows1_v, acc_sh, semi0, semi1, semr0, semr1):
    cid = lax.axis_index("c")
    sid = lax.axis_index("s")
    wid = cid * _NS + sid
    rows = (rows0_v, rows1_v)
    semi = (semi0, semi1)
    semr = (semr0, semr1)

    def _start_idx(blk, slot):
        pltpu.async_copy(eidx_hbm.at[wid, blk], cidx_v.at[slot], semi[slot])

    def _wait_idx(slot):
        pltpu.make_async_copy(eidx_hbm.at[0, 0], cidx_v.at[slot],
                              semi[slot]).wait()

    def _start_gather(slot, j, p):
        pltpu.async_copy(m_hbm.at[cidx_v.at[slot, j, 0]], rows[p], semr[p])

    def _wait_gather(p):
        pltpu.make_async_copy(m_hbm.at[pl.ds(0, _CH)], rows[p],
                              semr[p]).wait()

    def _scatter(slot, j, p):
        pltpu.sync_copy(rows[p], acc_sh.at[cidx_v.at[slot, j, 1]], add=True)

    def _do_block(blk_static, k, slot):
        for j in range(_BCH):
            g = blk_static * _BCH + j
            _wait_gather(g % 2)
            last = (k is None) and (slot == 0) and (j == _BCH - 1)
            if not last:
                nslot = (slot + (1 if j == _BCH - 1 else 0)) % 2
                if j == _BCH - 1:
                    _wait_idx(nslot)
                _start_gather(nslot, (j + 1) % _BCH, (g + 1) % 2)
            _scatter(slot, j, g % 2)

    _start_idx(0, 0)
    _start_idx(1, 1)
    pltpu.sync_copy(zeros_hbm.at[pl.ds(0, _RB)],
                    acc_sh.at[pl.ds(sid * _RB, _RB)])

    @pl.when(sid == _NS - 1)
    def _zero_tail():
        pltpu.sync_copy(zeros_hbm.at[pl.ds(0, _RREM)],
                        acc_sh.at[pl.ds(_NS * _RB, _RREM)])

    _wait_idx(0)
    _start_gather(0, 0, 0)
    plsc.subcore_barrier()

    def superblock(k, carry):
        _do_block(0, k, 0)
        _start_idx(2 * k + 2, 0)

        _do_block(1, k, 1)

        @pl.when(2 * k + 3 < _NBLK)
        def _pf():
            _start_idx(2 * k + 3, 1)

        return carry

    lax.fori_loop(0, (_NBLK - 1) // 2, superblock, 0)
    _do_block(0, None, 0)
    plsc.subcore_barrier()
    pltpu.sync_copy(acc_sh.at[pl.ds(sid * _RB, _RB)],
                    part_hbm.at[pl.ds(cid * _N + sid * _RB, _RB)])

    @pl.when(sid == _NS - 1)
    def _drain_tail():
        pltpu.sync_copy(acc_sh.at[pl.ds(_NS * _RB, _RREM)],
                        part_hbm.at[pl.ds(cid * _N + _NS * _RB, _RREM)])


@functools.partial(
    pl.kernel,
    out_type=jax.ShapeDtypeStruct((2 * _Q, _H), jnp.float32),
    mesh=_sc_mesh,
    scratch_types=[
        pltpu.VMEM((_QCH, 128), jnp.int32),
        pltpu.VMEM((128, _H), jnp.float32),
        pltpu.SemaphoreType.DMA,
    ],
)
def _egather(h_hbm, eidx_hbm, out_hbm, idx_v, rows_v, sem):
    wid = lax.axis_index("c") * _NS + lax.axis_index("s")
    pltpu.sync_copy(eidx_hbm.at[wid], idx_v)
    for j in range(_QCH):
        pltpu.async_copy(h_hbm.at[idx_v.at[j]], rows_v, sem).wait()
        pltpu.sync_copy(rows_v, out_hbm.at[pl.ds(wid * _QW + j * 128, 128)])


def _mm(a, b):
    return jnp.dot(a, b, preferred_element_type=jnp.float32)


def _relu_lin_body(x_ref, w_ref, b_ref, o_ref):
    o_ref[...] = jnp.maximum(_mm(x_ref[...], w_ref[...]) + b_ref[...], 0.0)


def _relu_lin(x, w, b):
    n, bn = x.shape[0], 1000
    return pl.pallas_call(
        _relu_lin_body,
        grid=(n // bn,),
        in_specs=[pl.BlockSpec((bn, x.shape[1]), lambda i: (i, 0)),
                  pl.BlockSpec(w.shape, lambda i: (0, 0)),
                  pl.BlockSpec((1, w.shape[1]), lambda i: (0, 0))],
        out_specs=pl.BlockSpec((bn, w.shape[1]), lambda i: (i, 0)),
        out_shape=jax.ShapeDtypeStruct((n, w.shape[1]), jnp.float32),
    )(x, w, b.reshape(1, -1))


def _update(p0, p1, h, wa, wb, b, h_out):
    aggr = p0[...] + p1[...]
    t = _mm(aggr, wa[...]) + _mm(h[...], wb[...]) + b[...]
    o = jnp.maximum(t, 0.0)
    nrm = jnp.sqrt(jnp.sum(o * o, axis=1, keepdims=True))
    o = o / jnp.maximum(nrm, 1e-12)
    h_out[...] = o
    return o


def _combine_m_body(p0, p1, h, wa, wb, b, wn, bn_, h_out, m_out):
    o = _update(p0, p1, h, wa, wb, b, h_out)
    m_out[...] = jnp.maximum(_mm(o, wn[...]) + bn_[...], 0.0)


def _combine_body(p0, p1, h, wa, wb, b, h_out):
    _update(p0, p1, h, wa, wb, b, h_out)


def _combine(p0, p1, h, w_agg, b_agg, w_next=None, b_next=None):
    n, bn = _N, 1000
    wa, wb = w_agg[:w_agg.shape[0] - _H], w_agg[w_agg.shape[0] - _H:]
    row = lambda i: (i, 0)
    full = lambda i: (0, 0)
    in_specs = [pl.BlockSpec((bn, _H), row), pl.BlockSpec((bn, _H), row),
                pl.BlockSpec((bn, h.shape[1]), row),
                pl.BlockSpec(wa.shape, full), pl.BlockSpec(wb.shape, full),
                pl.BlockSpec((1, _H), full)]
    args = [p0, p1, h, wa, wb, b_agg.reshape(1, -1)]
    out_spec = pl.BlockSpec((bn, _H), row)
    if w_next is None:
        return pl.pallas_call(
            _combine_body, grid=(n // bn,), in_specs=in_specs,
            out_specs=out_spec,
            out_shape=jax.ShapeDtypeStruct((n, _H), jnp.float32),
        )(*args)
    in_specs += [pl.BlockSpec(w_next.shape, full), pl.BlockSpec((1, _H), full)]
    args += [w_next, b_next.reshape(1, -1)]
    return pl.pallas_call(
        _combine_m_body, grid=(n // bn,), in_specs=in_specs,
        out_specs=(out_spec, out_spec),
        out_shape=(jax.ShapeDtypeStruct((n, _H), jnp.float32),
                   jax.ShapeDtypeStruct((n, _H), jnp.float32)),
    )(*args)


def _final_body(es, ed, w1a, w1b, b1, w2, b2, o_ref):
    z = _mm(es[...], w1a[...]) + _mm(ed[...], w1b[...]) + b1[...]
    z2 = _mm(z, w2[...]) + b2[...]
    m = jnp.max(z2, axis=1, keepdims=True)
    lse = m + jnp.log(jnp.sum(jnp.exp(z2 - m), axis=1, keepdims=True))
    o_ref[...] = z2 - lse


def _final(es, ed, w1, b1, w2, b2):
    bn = 1024
    row = lambda i: (i, 0)
    full = lambda i: (0, 0)
    return pl.pallas_call(
        _final_body,
        grid=(_Q // bn,),
        in_specs=[pl.BlockSpec((bn, _H), row), pl.BlockSpec((bn, _H), row),
                  pl.BlockSpec((_H, _H), full), pl.BlockSpec((_H, _H), full),
                  pl.BlockSpec((1, _H), full), pl.BlockSpec((_H, _O), full),
                  pl.BlockSpec((1, _O), full)],
        out_specs=pl.BlockSpec((bn, _O), row),
        out_shape=jax.ShapeDtypeStruct((_Q, _O), jnp.float32),
    )(es, ed, w1[:_H], w1[_H:], b1.reshape(1, -1), w2, b2.reshape(1, -1))


def kernel(x, edge_index, batch, eval_edges,
           lin_W0, lin_b0, agg_W0, agg_b0,
           lin_W1, lin_b1, agg_W1, agg_b1,
           mp_W1, mp_b1, mp_W2, mp_b2):
    ei = edge_index.reshape(2, _NW, _EPW)
    fill = jnp.array([0, _N], jnp.int32).reshape(2, 1, 1)
    ei = jnp.concatenate(
        [ei, jnp.broadcast_to(fill, (2, _NW, _EPWP - _EPW))], axis=2)
    eidx = ei.reshape(2, _NW, _NBLK, _BCH, _CH).transpose(1, 2, 3, 0, 4)
    zeros = jnp.zeros((_RB, _H), jnp.float32)

    m0 = _relu_lin(x, lin_W0, lin_b0)
    parts = _spmm(m0, eidx, zeros)
    h1, m1 = _combine(parts[:_N], parts[_N:], x, agg_W0, agg_b0,
                      w_next=lin_W1, b_next=lin_b1)
    parts = _spmm(m1, eidx, zeros)
    h2 = _combine(parts[:_N], parts[_N:], h1, agg_W1, agg_b1)

    eidx = jnp.concatenate([eval_edges[0], eval_edges[1]]).reshape(
        _NW, _QCH, 128)
    eo = _egather(h2, eidx)
    return _final(eo[:_Q], eo[_Q:], mp_W1, mp_b1, mp_W2, mp_b2)


# speedup vs baseline: 1.0544x; 1.0544x over previous
import functools

import jax
import jax.numpy as jnp
from jax import lax
from jax.experimental import pallas as pl
from jax.experimental.pallas import tpu as pltpu
from jax.experimental.pallas import tpu_sc as plsc

_N, _E, _D, _H, _O, _Q = 10000, 320000, 128, 128, 2, 4096
_NC, _NS = 2, 16
_NW = _NC * _NS
_EPW = _E // _NW
_CH = 80
_NCH = _EPW // _CH
_RB = 624
_RREM = _N - _NS * _RB
_QW = 2 * _Q // _NW
_QCH = _QW // 128

_sc_mesh = plsc.VectorSubcoreMesh(core_axis_name="c", subcore_axis_name="s")


@functools.partial(
    pl.kernel,
    out_type=jax.ShapeDtypeStruct((_NC * _N, _H), jnp.float32),
    mesh=_sc_mesh,
    scratch_types=[
        pltpu.VMEM((_NCH, _CH), jnp.int32),
        pltpu.VMEM((_NCH, _CH), jnp.int32),
        pltpu.VMEM((_CH, _H), jnp.float32),
        pltpu.VMEM_SHARED((_N, _H), jnp.float32),
        pltpu.SemaphoreType.DMA,
    ],
)
def _spmm(m_hbm, src_hbm, dst_hbm, zeros_hbm, part_hbm,
          src_v, dst_v, rows_v, acc_sh, sem):
    cid = lax.axis_index("c")
    sid = lax.axis_index("s")
    pltpu.sync_copy(src_hbm.at[cid * _NS + sid], src_v)
    pltpu.sync_copy(dst_hbm.at[cid * _NS + sid], dst_v)
    pltpu.sync_copy(zeros_hbm.at[pl.ds(0, _RB)],
                    acc_sh.at[pl.ds(sid * _RB, _RB)])

    @pl.when(sid == _NS - 1)
    def _zero_tail():
        pltpu.sync_copy(zeros_hbm.at[pl.ds(0, _RREM)],
                        acc_sh.at[pl.ds(_NS * _RB, _RREM)])

    plsc.subcore_barrier()

    def step(i, carry):
        pltpu.async_copy(m_hbm.at[src_v.at[i]], rows_v, sem).wait()
        pltpu.sync_copy(rows_v, acc_sh.at[dst_v.at[i]], add=True)
        return carry

    lax.fori_loop(0, _NCH, step, 0)
    plsc.subcore_barrier()
    pltpu.sync_copy(acc_sh.at[pl.ds(sid * _RB, _RB)],
                    part_hbm.at[pl.ds(cid * _N + sid * _RB, _RB)])

    @pl.when(sid == _NS - 1)
    def _drain_tail():
        pltpu.sync_copy(acc_sh.at[pl.ds(_NS * _RB, _RREM)],
                        part_hbm.at[pl.ds(cid * _N + _NS * _RB, _RREM)])


@functools.partial(
    pl.kernel,
    out_type=jax.ShapeDtypeStruct((2 * _Q, _H), jnp.float32),
    mesh=_sc_mesh,
    scratch_types=[
        pltpu.VMEM((_QCH, 128), jnp.int32),
        pltpu.VMEM((128, _H), jnp.float32),
        pltpu.SemaphoreType.DMA,
    ],
)
def _egather(h_hbm, eidx_hbm, out_hbm, idx_v, rows_v, sem):
    wid = lax.axis_index("c") * _NS + lax.axis_index("s")
    pltpu.sync_copy(eidx_hbm.at[wid], idx_v)
    for j in range(_QCH):
        pltpu.async_copy(h_hbm.at[idx_v.at[j]], rows_v, sem).wait()
        pltpu.sync_copy(rows_v, out_hbm.at[pl.ds(wid * _QW + j * 128, 128)])


def _mm(a, b):
    return jnp.dot(a, b, preferred_element_type=jnp.float32)


def _relu_lin_body(x_ref, w_ref, b_ref, o_ref):
    o_ref[...] = jnp.maximum(_mm(x_ref[...], w_ref[...]) + b_ref[...], 0.0)


def _relu_lin(x, w, b):
    n, bn = x.shape[0], 1000
    return pl.pallas_call(
        _relu_lin_body,
        grid=(n // bn,),
        in_specs=[pl.BlockSpec((bn, x.shape[1]), lambda i: (i, 0)),
                  pl.BlockSpec(w.shape, lambda i: (0, 0)),
                  pl.BlockSpec((1, w.shape[1]), lambda i: (0, 0))],
        out_specs=pl.BlockSpec((bn, w.shape[1]), lambda i: (i, 0)),
        out_shape=jax.ShapeDtypeStruct((n, w.shape[1]), jnp.float32),
    )(x, w, b.reshape(1, -1))


def _update(p0, p1, h, wa, wb, b, h_out):
    aggr = p0[...] + p1[...]
    t = _mm(aggr, wa[...]) + _mm(h[...], wb[...]) + b[...]
    o = jnp.maximum(t, 0.0)
    nrm = jnp.sqrt(jnp.sum(o * o, axis=1, keepdims=True))
    o = o / jnp.maximum(nrm, 1e-12)
    h_out[...] = o
    return o


def _combine_m_body(p0, p1, h, wa, wb, b, wn, bn_, h_out, m_out):
    o = _update(p0, p1, h, wa, wb, b, h_out)
    m_out[...] = jnp.maximum(_mm(o, wn[...]) + bn_[...], 0.0)


def _combine_body(p0, p1, h, wa, wb, b, h_out):
    _update(p0, p1, h, wa, wb, b, h_out)


def _combine(p0, p1, h, w_agg, b_agg, w_next=None, b_next=None):
    n, bn = _N, 1000
    wa, wb = w_agg[:w_agg.shape[0] - _H], w_agg[w_agg.shape[0] - _H:]
    row = lambda i: (i, 0)
    full = lambda i: (0, 0)
    in_specs = [pl.BlockSpec((bn, _H), row), pl.BlockSpec((bn, _H), row),
                pl.BlockSpec((bn, h.shape[1]), row),
                pl.BlockSpec(wa.shape, full), pl.BlockSpec(wb.shape, full),
                pl.BlockSpec((1, _H), full)]
    args = [p0, p1, h, wa, wb, b_agg.reshape(1, -1)]
    out_spec = pl.BlockSpec((bn, _H), row)
    if w_next is None:
        return pl.pallas_call(
            _combine_body, grid=(n // bn,), in_specs=in_specs,
            out_specs=out_spec,
            out_shape=jax.ShapeDtypeStruct((n, _H), jnp.float32),
        )(*args)
    in_specs += [pl.BlockSpec(w_next.shape, full), pl.BlockSpec((1, _H), full)]
    args += [w_next, b_next.reshape(1, -1)]
    return pl.pallas_call(
        _combine_m_body, grid=(n // bn,), in_specs=in_specs,
        out_specs=(out_spec, out_spec),
        out_shape=(jax.ShapeDtypeStruct((n, _H), jnp.float32),
                   jax.ShapeDtypeStruct((n, _H), jnp.float32)),
    )(*args)


def _final_body(es, ed, w1a, w1b, b1, w2, b2, o_ref):
    z = _mm(es[...], w1a[...]) + _mm(ed[...], w1b[...]) + b1[...]
    z2 = _mm(z, w2[...]) + b2[...]
    m = jnp.max(z2, axis=1, keepdims=True)
    lse = m + jnp.log(jnp.sum(jnp.exp(z2 - m), axis=1, keepdims=True))
    o_ref[...] = z2 - lse


def _final(es, ed, w1, b1, w2, b2):
    bn = 1024
    row = lambda i: (i, 0)
    full = lambda i: (0, 0)
    return pl.pallas_call(
        _final_body,
        grid=(_Q // bn,),
        in_specs=[pl.BlockSpec((bn, _H), row), pl.BlockSpec((bn, _H), row),
                  pl.BlockSpec((_H, _H), full), pl.BlockSpec((_H, _H), full),
                  pl.BlockSpec((1, _H), full), pl.BlockSpec((_H, _O), full),
                  pl.BlockSpec((1, _O), full)],
        out_specs=pl.BlockSpec((bn, _O), row),
        out_shape=jax.ShapeDtypeStruct((_Q, _O), jnp.float32),
    )(es, ed, w1[:_H], w1[_H:], b1.reshape(1, -1), w2, b2.reshape(1, -1))


def kernel(x, edge_index, batch, eval_edges,
           lin_W0, lin_b0, agg_W0, agg_b0,
           lin_W1, lin_b1, agg_W1, agg_b1,
           mp_W1, mp_b1, mp_W2, mp_b2):
    src = edge_index[0].reshape(_NW, _NCH, _CH)
    dst = edge_index[1].reshape(_NW, _NCH, _CH)
    zeros = jnp.zeros((_RB, _H), jnp.float32)

    m0 = _relu_lin(x, lin_W0, lin_b0)
    parts = _spmm(m0, src, dst, zeros)
    h1, m1 = _combine(parts[:_N], parts[_N:], x, agg_W0, agg_b0,
                      w_next=lin_W1, b_next=lin_b1)
    parts = _spmm(m1, src, dst, zeros)
    h2 = _combine(parts[:_N], parts[_N:], h1, agg_W1, agg_b1)

    eidx = jnp.concatenate([eval_edges[0], eval_edges[1]]).reshape(
        _NW, _QCH, 128)
    eo = _egather(h2, eidx)
    return _final(eo[:_Q], eo[_Q:], mp_W1, mp_b1, mp_W2, mp_b2)
